# baseline (speedup 1.0000x reference)
import functools

import jax
import jax.numpy as jnp
from jax import lax
from jax.experimental import pallas as pl
from jax.experimental.pallas import tpu as pltpu
from jax.experimental.pallas import tpu_sc as plsc

N = 256
D = 32
K_NEIGHBORS = 50
NSTEPS = 20
GAMMA = 0.1
DT = 0.05
BATCH = 4
H1 = 16
BD = BATCH * D
BH = BATCH * H1

SC_CORES = 2
SC_SUBCORES = 16
LANES = 16
WORKERS = SC_CORES * SC_SUBCORES
ROWS_W = N // WORKERS
NCH = N // LANES

_PREC = jax.lax.Precision.HIGHEST
_F32 = jnp.float32
_I32 = jnp.int32


def _sim_body(feat_ref, sim_ref):
    feat = feat_ref[...]
    nrm = jnp.sqrt(jnp.sum(feat * feat, axis=1, keepdims=True))
    fn = feat / jnp.maximum(nrm, 1e-12)
    sim_ref[...] = jax.lax.dot_general(
        fn, fn, (((1,), (1,)), ((), ())),
        precision=_PREC, preferred_element_type=_F32)


_sc_mesh = plsc.VectorSubcoreMesh(core_axis_name="c", subcore_axis_name="s")


@functools.partial(
    pl.kernel,
    mesh=_sc_mesh,
    out_type=jax.ShapeDtypeStruct((N, N), jnp.float32),
    compiler_params=pltpu.CompilerParams(needs_layout_passes=False),
)
def _sc_topk(sim_hbm, nmat_hbm):
    wid = lax.axis_index("s") * SC_CORES + lax.axis_index("c")
    base = wid * ROWS_W

    def scoped(rows_v, keys_v, out_v):
        pltpu.sync_copy(sim_hbm.at[pl.ds(base, ROWS_W)], rows_v)

        one = jnp.full((LANES,), 1, _I32)
        zero = jnp.zeros((LANES,), _I32)
        signbit = jnp.full((LANES,), -2147483648, _I32)

        for r in range(ROWS_W):
            for c in range(NCH):
                f = rows_v[r, pl.ds(c * LANES, LANES)]
                ib = lax.bitcast_convert_type(f, _I32)
                key = jnp.where(ib < 0, jnp.bitwise_xor(~ib, signbit), ib)
                keys_v[r, pl.ds(c * LANES, LANES)] = key

            def bis(_, carry, r=r):
                lo, hi = carry
                mid = lo + lax.shift_right_logical(hi - lo, 1)
                cnt = _I32(0)
                for c in range(NCH):
                    k = keys_v[r, pl.ds(c * LANES, LANES)]
                    cnt = cnt + jnp.sum(jnp.where(k >= mid, one, zero))
                ge = cnt >= K_NEIGHBORS
                return jnp.where(ge, mid, lo), jnp.where(ge, hi, mid)

            v50, _ = lax.fori_loop(
                0, 31, bis, (_I32(-(2 ** 30)), _I32(2 ** 30 - 1)))

            gtc = _I32(0)
            for c in range(NCH):
                k = keys_v[r, pl.ds(c * LANES, LANES)]
                gtc = gtc + jnp.sum(jnp.where(k > v50, one, zero))
            need = K_NEIGHBORS - gtc

            run = _I32(0)
            for c in range(NCH):
                k = keys_v[r, pl.ds(c * LANES, LANES)]
                tie = k == v50
                ti = jnp.where(tie, one, zero)
                excl = plsc.cumsum(ti) - ti
                keep = tie & ((run + excl) < need)
                sel = (k > v50) | keep
                f = rows_v[r, pl.ds(c * LANES, LANES)]
                out_v[r, pl.ds(c * LANES, LANES)] = jnp.where(sel, f, 0.0)
                run = run + jnp.sum(ti)

        pltpu.sync_copy(out_v, nmat_hbm.at[pl.ds(base, ROWS_W)])

    pl.run_scoped(
        scoped,
        pltpu.VMEM((ROWS_W, N), jnp.float32),
        pltpu.VMEM((ROWS_W, N), jnp.int32),
        pltpu.VMEM((ROWS_W, N), jnp.float32),
    )


def _dyn_body(u_ref, nmat_ref, bias_ref, in_wT_ref, in_b_ref, m1_ref,
              s1b_ref, m2_ref, s2b_ref, mred_ref, x0_ref, out_ref):
    nmat = nmat_ref[...]

    in_wT = in_wT_ref[...]
    ups = [jax.lax.dot_general(u_ref[b], in_wT, (((1,), (0,)), ((), ())),
                               precision=_PREC, preferred_element_type=_F32)
           for b in range(BATCH)]
    bias = bias_ref[...]
    const = jnp.concatenate([up + bias for up in ups], axis=1) + in_b_ref[...]

    m1 = m1_ref[...]
    s1b = s1b_ref[...]
    m2 = m2_ref[...]
    s2b = s2b_ref[...]
    mred = mred_ref[...]

    x_init = jnp.concatenate([x0_ref[b] for b in range(BATCH)], axis=1)

    inv_sqrt2 = _F32(0.7071067811865476)

    def step(_, x):
        sq = jax.lax.dot_general(x * x, mred, (((1,), (0,)), ((), ())),
                                 precision=_PREC,
                                 preferred_element_type=_F32)
        amat = jnp.tanh(jnp.sqrt(sq + 1e-12))
        syn = jax.lax.dot_general(nmat, amat, (((1,), (0,)), ((), ())),
                                  precision=_PREC,
                                  preferred_element_type=_F32)
        pre = jax.lax.dot_general(syn, m1, (((1,), (0,)), ((), ())),
                                  precision=_PREC,
                                  preferred_element_type=_F32) + s1b
        h1 = 0.5 * pre * (1.0 + jax.lax.erf(pre * inv_sqrt2))
        sig = jax.lax.dot_general(h1, m2, (((1,), (0,)), ((), ())),
                                  precision=_PREC,
                                  preferred_element_type=_F32) + s2b
        return x + (sig + const - GAMMA * x) * DT

    x = jax.lax.fori_loop(0, NSTEPS, step, x_init)
    for b in range(BATCH):
        out_ref[b] = x[:, b * D:(b + 1) * D]


def kernel(u, features, bias, W, in_w, in_b, s1_w, s1_b, s2_w, s2_b, x0,
           a_bar0):
    del W, a_bar0

    sim = pl.pallas_call(
        _sim_body,
        out_shape=jax.ShapeDtypeStruct((N, N), jnp.float32),
    )(features)

    nmat = _sc_topk(sim)

    eye_b = jnp.eye(BATCH, dtype=jnp.float32)
    m1 = jnp.kron(eye_b, s1_w.reshape(1, H1))
    m2 = jnp.kron(eye_b, s2_w.T)
    mred = jnp.kron(eye_b, jnp.ones((D, 1), jnp.float32))
    return pl.pallas_call(
        _dyn_body,
        out_shape=jax.ShapeDtypeStruct((BATCH, N, D), jnp.float32),
    )(u, nmat, bias, in_w.T, jnp.tile(in_b, BATCH).reshape(1, BD),
      m1, jnp.tile(s1_b, BATCH).reshape(1, BH), m2,
      jnp.tile(s2_b, BATCH).reshape(1, BD), mred, x0)


# speedup vs baseline: 1.0160x; 1.0160x over previous
import functools

import jax
import jax.numpy as jnp
from jax import lax
from jax.experimental import pallas as pl
from jax.experimental.pallas import tpu as pltpu
from jax.experimental.pallas import tpu_sc as plsc

N = 256
D = 32
K_NEIGHBORS = 50
NSTEPS = 20
GAMMA = 0.1
DT = 0.05
BATCH = 4
H1 = 16
BD = BATCH * D
BH = BATCH * H1

SC_CORES = 2
SC_SUBCORES = 16
LANES = 16
WORKERS = SC_CORES * SC_SUBCORES
ROWS_W = N // WORKERS
NCH = N // LANES

_PREC = jax.lax.Precision.HIGHEST
_F32 = jnp.float32
_I32 = jnp.int32


def _sim_body(feat_ref, sim_ref):
    feat = feat_ref[...]
    nrm = jnp.sqrt(jnp.sum(feat * feat, axis=1, keepdims=True))
    fn = feat / jnp.maximum(nrm, 1e-12)
    sim_ref[...] = jax.lax.dot_general(
        fn, fn, (((1,), (1,)), ((), ())),
        precision=_PREC, preferred_element_type=_F32)


_sc_mesh = plsc.VectorSubcoreMesh(core_axis_name="c", subcore_axis_name="s")


@functools.partial(
    pl.kernel,
    mesh=_sc_mesh,
    out_type=jax.ShapeDtypeStruct((N, N), jnp.float32),
    compiler_params=pltpu.CompilerParams(needs_layout_passes=False),
)
def _sc_topk(sim_hbm, nmat_hbm):
    wid = lax.axis_index("s") * SC_CORES + lax.axis_index("c")
    base = wid * ROWS_W

    def scoped(rows_v, out_v):
        pltpu.sync_copy(sim_hbm.at[pl.ds(base, ROWS_W)], rows_v)

        one = jnp.full((LANES,), 1, _I32)
        zero = jnp.zeros((LANES,), _I32)
        signbit = jnp.full((LANES,), -2147483648, _I32)

        for r in range(ROWS_W):
            ks = []
            for c in range(NCH):
                f = rows_v[r, pl.ds(c * LANES, LANES)]
                ib = lax.bitcast_convert_type(f, _I32)
                ks.append(jnp.where(ib < 0, jnp.bitwise_xor(~ib, signbit),
                                    ib))
            ks = tuple(ks)

            def bis(_, carry):
                lo, hi, kk = carry
                mid = lo + lax.shift_right_logical(hi - lo, 1)
                cv = zero
                for k in kk:
                    cv = cv + jnp.where(k >= mid, one, zero)
                ge = jnp.sum(cv) >= K_NEIGHBORS
                return (jnp.where(ge, mid, lo), jnp.where(ge, hi, mid), kk)

            v50, _, _ = lax.fori_loop(
                0, 31, bis, (_I32(-(2 ** 30)), _I32(2 ** 30 - 1), ks))

            gv = zero
            for k in ks:
                gv = gv + jnp.where(k > v50, one, zero)
            need = K_NEIGHBORS - jnp.sum(gv)

            run = _I32(0)
            for c in range(NCH):
                k = ks[c]
                tie = k == v50
                ti = jnp.where(tie, one, zero)
                excl = plsc.cumsum(ti) - ti
                keep = tie & ((run + excl) < need)
                sel = (k > v50) | keep
                f = rows_v[r, pl.ds(c * LANES, LANES)]
                out_v[r, pl.ds(c * LANES, LANES)] = jnp.where(sel, f, 0.0)
                run = run + jnp.sum(ti)

        pltpu.sync_copy(out_v, nmat_hbm.at[pl.ds(base, ROWS_W)])

    pl.run_scoped(
        scoped,
        pltpu.VMEM((ROWS_W, N), jnp.float32),
        pltpu.VMEM((ROWS_W, N), jnp.float32),
    )


def _dyn_body(u_ref, nmat_ref, bias_ref, in_wT_ref, in_b_ref, m1_ref,
              s1b_ref, m2_ref, s2b_ref, mred_ref, x0_ref, out_ref):
    nmat = nmat_ref[...]

    in_wT = in_wT_ref[...]
    ups = [jax.lax.dot_general(u_ref[b], in_wT, (((1,), (0,)), ((), ())),
                               precision=_PREC, preferred_element_type=_F32)
           for b in range(BATCH)]
    bias = bias_ref[...]
    const = jnp.concatenate([up + bias for up in ups], axis=1) + in_b_ref[...]

    m1 = m1_ref[...]
    s1b = s1b_ref[...]
    m2 = m2_ref[...]
    s2b = s2b_ref[...]
    mred = mred_ref[...]

    x_init = jnp.concatenate([x0_ref[b] for b in range(BATCH)], axis=1)

    inv_sqrt2 = _F32(0.7071067811865476)

    def step(_, x):
        sq = jax.lax.dot_general(x * x, mred, (((1,), (0,)), ((), ())),
                                 precision=_PREC,
                                 preferred_element_type=_F32)
        amat = jnp.tanh(jnp.sqrt(sq + 1e-12))
        syn = jax.lax.dot_general(nmat, amat, (((1,), (0,)), ((), ())),
                                  precision=_PREC,
                                  preferred_element_type=_F32)
        pre = jax.lax.dot_general(syn, m1, (((1,), (0,)), ((), ())),
                                  precision=_PREC,
                                  preferred_element_type=_F32) + s1b
        h1 = 0.5 * pre * (1.0 + jax.lax.erf(pre * inv_sqrt2))
        sig = jax.lax.dot_general(h1, m2, (((1,), (0,)), ((), ())),
                                  precision=_PREC,
                                  preferred_element_type=_F32) + s2b
        return x + (sig + const - GAMMA * x) * DT

    x = jax.lax.fori_loop(0, NSTEPS, step, x_init)
    for b in range(BATCH):
        out_ref[b] = x[:, b * D:(b + 1) * D]


def kernel(u, features, bias, W, in_w, in_b, s1_w, s1_b, s2_w, s2_b, x0,
           a_bar0):
    del W, a_bar0

    sim = pl.pallas_call(
        _sim_body,
        out_shape=jax.ShapeDtypeStruct((N, N), jnp.float32),
    )(features)

    nmat = _sc_topk(sim)

    eye_b = jnp.eye(BATCH, dtype=jnp.float32)
    m1 = jnp.kron(eye_b, s1_w.reshape(1, H1))
    m2 = jnp.kron(eye_b, s2_w.T)
    mred = jnp.kron(eye_b, jnp.ones((D, 1), jnp.float32))
    return pl.pallas_call(
        _dyn_body,
        out_shape=jax.ShapeDtypeStruct((BATCH, N, D), jnp.float32),
    )(u, nmat, bias, in_w.T, jnp.tile(in_b, BATCH).reshape(1, BD),
      m1, jnp.tile(s1_b, BATCH).reshape(1, BH), m2,
      jnp.tile(s2_b, BATCH).reshape(1, BD), mred, x0)
